# Initial kernel scaffold; baseline (speedup 1.0000x reference)
import jax
import jax.numpy as jnp
from jax.experimental import pallas as pl


def kernel(tag_ids, embedding_table):
    raise NotImplementedError("write your pallas kernel here")


# speedup vs baseline: 1.4781x; 1.4781x over previous
import functools

import jax
import jax.numpy as jnp
from jax import lax
from jax.experimental import pallas as pl
from jax.experimental.pallas import tpu as pltpu
from jax.experimental.pallas import tpu_sc as plsc

BATCH = 4096
HIST = 200
D = 32
TOTAL = BATCH * HIST

NC = 2
NS = 16
NW = NC * NS
ROWS_PER_W = TOTAL // NW
STREAM = 128
K = 8
CHUNK = STREAM * K
NCHUNK = ROWS_PER_W // CHUNK
NSTREAM = ROWS_PER_W // STREAM


@jax.jit
def _gather(tag_ids_r, table):
    mesh = plsc.VectorSubcoreMesh(core_axis_name="c", subcore_axis_name="s")

    @functools.partial(
        pl.kernel,
        mesh=mesh,
        out_type=jax.ShapeDtypeStruct((NW, NCHUNK, K, STREAM, D), jnp.float32),
        scratch_types=[
            pltpu.VMEM((NSTREAM, STREAM), jnp.int32),
            pltpu.VMEM((K, STREAM, D), jnp.float32),
            pltpu.SemaphoreType.DMA,
        ],
        compiler_params=pltpu.CompilerParams(use_tc_tiling_on_sc=False),
    )
    def k(idx_hbm, table_hbm, out_hbm, idx_v, rows_v, sem):
        wid = lax.axis_index("s") * NC + lax.axis_index("c")
        pltpu.sync_copy(idx_hbm.at[wid], idx_v)

        def chunk_body(c, carry):
            cps = [
                pltpu.async_copy(
                    table_hbm.at[idx_v.at[c * K + b]], rows_v.at[b], sem
                )
                for b in range(K)
            ]
            for cp in cps:
                cp.wait()
            pltpu.sync_copy(rows_v, out_hbm.at[wid, c])
            return carry

        lax.fori_loop(0, NCHUNK, chunk_body, 0)

    return k(tag_ids_r, table)


def kernel(tag_ids, embedding_table):
    idx = tag_ids.reshape(NW, NSTREAM, STREAM).astype(jnp.int32)
    out = _gather(idx, embedding_table)
    return out.reshape(BATCH, HIST, D)


# speedup vs baseline: 1.5002x; 1.0150x over previous
import functools

import jax
import jax.numpy as jnp
from jax import lax
from jax.experimental import pallas as pl
from jax.experimental.pallas import tpu as pltpu
from jax.experimental.pallas import tpu_sc as plsc

BATCH = 4096
HIST = 200
D = 32
TOTAL = BATCH * HIST

NC = 2
NS = 16
NW = NC * NS
ROWS_PER_W = TOTAL // NW
STREAM = 128
K = 10
CHUNK = STREAM * K
NCHUNK = ROWS_PER_W // CHUNK
NSTREAM = ROWS_PER_W // STREAM


@jax.jit
def _gather(tag_ids_r, table):
    mesh = plsc.VectorSubcoreMesh(core_axis_name="c", subcore_axis_name="s")

    @functools.partial(
        pl.kernel,
        mesh=mesh,
        out_type=jax.ShapeDtypeStruct((NW, NCHUNK, K, STREAM, D), jnp.float32),
        scratch_types=[
            pltpu.VMEM((NSTREAM, STREAM), jnp.int32),
            pltpu.VMEM((2, K, STREAM, D), jnp.float32),
            pltpu.SemaphoreType.DMA,
            pltpu.SemaphoreType.DMA,
            pltpu.SemaphoreType.DMA,
            pltpu.SemaphoreType.DMA,
        ],
        compiler_params=pltpu.CompilerParams(use_tc_tiling_on_sc=False),
    )
    def k(idx_hbm, table_hbm, out_hbm, idx_v, rows_v, gsem0, gsem1, wsem0, wsem1):
        wid = lax.axis_index("s") * NC + lax.axis_index("c")
        pltpu.sync_copy(idx_hbm.at[wid], idx_v)

        bufs = (rows_v.at[0], rows_v.at[1])
        gsems = (gsem0, gsem1)
        wsems = (wsem0, wsem1)

        def fire_gathers(c, i):
            for b in range(K):
                pltpu.async_copy(
                    table_hbm.at[idx_v.at[c * K + b]], bufs[i].at[b], gsems[i]
                )

        def drain_gathers(i):
            for b in range(K):
                pltpu.make_async_copy(
                    table_hbm.at[idx_v.at[b]], bufs[i].at[b], gsems[i]
                ).wait()

        def fire_write(c, i):
            pltpu.async_copy(bufs[i], out_hbm.at[wid, c], wsems[i])

        def drain_write(i):
            pltpu.make_async_copy(bufs[i], out_hbm.at[wid, 0], wsems[i]).wait()

        fire_gathers(0, 0)

        def body(c, carry):
            def step(i):
                nxt = 1 - i

                @pl.when(c >= 1)
                def _():
                    drain_write(nxt)

                @pl.when(c + 1 < NCHUNK)
                def _():
                    fire_gathers(c + 1, nxt)

                drain_gathers(i)
                fire_write(c, i)

            is_even = lax.rem(c, 2) == 0

            @pl.when(is_even)
            def _():
                step(0)

            @pl.when(jnp.logical_not(is_even))
            def _():
                step(1)

            return carry

        lax.fori_loop(0, NCHUNK, body, 0)
        drain_write((NCHUNK - 1) % 2)

    return k(tag_ids_r, table)


def kernel(tag_ids, embedding_table):
    idx = tag_ids.reshape(NW, NSTREAM, STREAM).astype(jnp.int32)
    out = _gather(idx, embedding_table)
    return out.reshape(BATCH, HIST, D)
